# baseline (speedup 1.0000x reference)
import functools

import jax
import jax.numpy as jnp
from jax.experimental import pallas as pl
from jax.experimental.pallas import tpu as pltpu

_BLK = 2000


def _a_body(d, x_ref, plef_ref, dep_ref, st_ref, w0a_ref, w0b_ref, u0a_ref,
            u0b_ref, u0_ref, u1w_ref, u1b_ref, v0b_ref, y_ref, z_ref):
    dep = dep_ref[...]
    st = st_ref[...]
    xb = x_ref[...]
    am = ((dep == d) & (st == 0)).astype(jnp.float32)
    bm = ((dep == d) & (st == 1)).astype(jnp.float32)
    hm = ((dep == d) & (st == 3)).astype(jnp.float32)
    y_ref[...] = (
        jnp.dot(xb * am, w0a_ref[...], preferred_element_type=jnp.float32)
        + jnp.dot(xb * bm, w0b_ref[...], preferred_element_type=jnp.float32))
    lh = jnp.maximum(
        jnp.dot(xb, u0a_ref[...], preferred_element_type=jnp.float32)
        + jnp.dot(plef_ref[...], u0b_ref[...], preferred_element_type=jnp.float32)
        + u0_ref[...], 0.0)
    ph = (jnp.dot(lh, u1w_ref[...], preferred_element_type=jnp.float32)
          + u1b_ref[...]) * hm
    z_ref[...] = jnp.dot(ph, v0b_ref[...], preferred_element_type=jnp.float32)


def _bm_body(d, blk, parents_sm, cd_sm, y_ref, m_ref, lc_ref):
    k = pl.program_id(0)

    @pl.when(k == 0)
    def _():
        m_ref[...] = jnp.zeros_like(m_ref)
        lc_ref[...] = jnp.zeros_like(lc_ref)

    base = k * blk

    def body(j, carry):
        i = base + j
        c = cd_sm[i]
        idx = parents_sm[i]

        @pl.when((c == 4 * d) | (c == 4 * d + 1))
        def _():
            m_ref[pl.ds(idx, 1), :] += y_ref[pl.ds(j, 1), :]

        @pl.when(c == 4 * d)
        def _():
            lc_ref[pl.ds(idx, 1), :] += 1.0

        return carry

    jax.lax.fori_loop(0, blk, body, 0)


def _bz_body(d, blk, parents_sm, cd_sm, z_ref, zacc_ref, hc_ref):
    k = pl.program_id(0)

    @pl.when(k == 0)
    def _():
        zacc_ref[...] = jnp.zeros_like(zacc_ref)
        hc_ref[...] = jnp.zeros_like(hc_ref)

    base = k * blk

    def body(j, carry):
        i = base + j
        c = cd_sm[i]
        idx = parents_sm[i]

        @pl.when(c == 4 * d + 3)
        def _():
            zacc_ref[pl.ds(idx, 1), :] += z_ref[pl.ds(j, 1), :]
            hc_ref[pl.ds(idx, 1), :] += 1.0

        return carry

    jax.lax.fori_loop(0, blk, body, 0)


def _c_body(x_ref, pef_ref, m_ref, zacc_ref, lc_ref, hc_ref, w0c_ref, b0_ref,
            w1_ref, b1_ref, v0a_ref, c0_ref, v1_ref, c1_ref, out_ref):
    xb = x_ref[...]
    pre1 = (m_ref[...]
            + jnp.dot(pef_ref[...], w0c_ref[...],
                      preferred_element_type=jnp.float32) + b0_ref[...])
    x_par = (jnp.dot(jnp.maximum(pre1, 0.0), w1_ref[...],
                     preferred_element_type=jnp.float32) + b1_ref[...])
    desg = hc_ref[...] != 0.0
    pm = lc_ref[...] != 0.0
    xd = jnp.where(desg, xb, 0.0)
    pre2 = (jnp.dot(xd, v0a_ref[...], preferred_element_type=jnp.float32)
            + zacc_ref[...] + c0_ref[...])
    x_mer = (jnp.dot(jnp.maximum(pre2, 0.0), v1_ref[...],
                     preferred_element_type=jnp.float32) + c1_ref[...])
    out_ref[...] = jnp.where(pm, x_par, jnp.where(desg, x_mer, xb))


def _row_spec(h):
    return pl.BlockSpec((_BLK, h), lambda k: (k, 0))


def _col_spec():
    return pl.BlockSpec((_BLK, 1), lambda k: (k, 0))


def _full_spec(shape):
    return pl.BlockSpec(shape, lambda k: tuple(0 for _ in shape))


def kernel(x, edge_index, depths, states, parent_edge_features,
           parent_light_edge_features, merger_params, lep_params, lem_params):
    n, h = x.shape
    nblk = n // _BLK
    (w0, b0), (w1, b1) = merger_params
    (u0w, u0b), (u1w, u1b) = lep_params
    (v0w, c0), (v1w, c1) = lem_params
    w0a, w0b, w0c = w0[:h], w0[h:2 * h], w0[2 * h:]
    u0a, u0bb = u0w[:h], u0w[h:]
    v0a, v0b = v0w[:h], v0w[h:]

    parents = jnp.zeros((n,), dtype=edge_index.dtype).at[edge_index[0]].set(
        edge_index[1])
    cd = (depths * 4 + states).astype(jnp.int32)
    dep2 = depths.reshape(n, 1)
    st2 = states.reshape(n, 1)

    f32 = jnp.float32
    wspec = _full_spec((h, h))
    bspec = _full_spec((h,))

    for d in (3, 2, 1):
        a_call = pl.pallas_call(
            functools.partial(_a_body, d),
            grid=(nblk,),
            in_specs=[_row_spec(h), _row_spec(h), _col_spec(), _col_spec(),
                      wspec, wspec, wspec, wspec, bspec, wspec, bspec, wspec],
            out_specs=[_row_spec(h), _row_spec(h)],
            out_shape=[jax.ShapeDtypeStruct((n, h), f32),
                       jax.ShapeDtypeStruct((n, h), f32)],
        )
        y, z = a_call(x, parent_light_edge_features, dep2, st2,
                      w0a, w0b, u0a, u0bb, u0b, u1w, u1b, v0b)

        def _scatter_spec():
            return pltpu.PrefetchScalarGridSpec(
                num_scalar_prefetch=2,
                grid=(nblk,),
                in_specs=[pl.BlockSpec((_BLK, h), lambda k, *_: (k, 0))],
                out_specs=[pl.BlockSpec((n, h), lambda k, *_: (0, 0)),
                           pl.BlockSpec((n, 1), lambda k, *_: (0, 0))],
            )

        m, lc = pl.pallas_call(
            functools.partial(_bm_body, d, _BLK),
            grid_spec=_scatter_spec(),
            out_shape=[jax.ShapeDtypeStruct((n, h), f32),
                       jax.ShapeDtypeStruct((n, 1), f32)],
        )(parents, cd, y)
        zacc, hc = pl.pallas_call(
            functools.partial(_bz_body, d, _BLK),
            grid_spec=_scatter_spec(),
            out_shape=[jax.ShapeDtypeStruct((n, h), f32),
                       jax.ShapeDtypeStruct((n, 1), f32)],
        )(parents, cd, z)

        c_call = pl.pallas_call(
            _c_body,
            grid=(nblk,),
            in_specs=[_row_spec(h), _row_spec(h), _row_spec(h), _row_spec(h),
                      _col_spec(), _col_spec(),
                      wspec, bspec, wspec, bspec, wspec, bspec, wspec, bspec],
            out_specs=_row_spec(h),
            out_shape=jax.ShapeDtypeStruct((n, h), f32),
        )
        x = c_call(x, parent_edge_features, m, zacc, lc, hc,
                   w0c, b0, w1, b1, v0a, c0, v1w, c1)
    return x


# speedup vs baseline: 1.3892x; 1.3892x over previous
import functools

import jax
import jax.numpy as jnp
from jax.experimental import pallas as pl
from jax.experimental.pallas import tpu as pltpu

_BLK = 2000


def _a_body(d, x_ref, plef_ref, dep_ref, st_ref, w0a_ref, w0b_ref, u0a_ref,
            u0b_ref, u0_ref, u1w_ref, u1b_ref, v0b_ref, y_ref, z_ref):
    dep = dep_ref[...]
    st = st_ref[...]
    xb = x_ref[...]
    am = ((dep == d) & (st == 0)).astype(jnp.float32)
    bm = ((dep == d) & (st == 1)).astype(jnp.float32)
    hm = ((dep == d) & (st == 3)).astype(jnp.float32)
    y_ref[...] = (
        jnp.dot(xb * am, w0a_ref[...], preferred_element_type=jnp.float32)
        + jnp.dot(xb * bm, w0b_ref[...], preferred_element_type=jnp.float32))
    lh = jnp.maximum(
        jnp.dot(xb, u0a_ref[...], preferred_element_type=jnp.float32)
        + jnp.dot(plef_ref[...], u0b_ref[...], preferred_element_type=jnp.float32)
        + u0_ref[...], 0.0)
    ph = (jnp.dot(lh, u1w_ref[...], preferred_element_type=jnp.float32)
          + u1b_ref[...]) * hm
    z_ref[...] = jnp.dot(ph, v0b_ref[...], preferred_element_type=jnp.float32)


def _compact_body(d, blk, cd_sm, ml_ref, hl_ref, mcnt_ref, hcnt_ref, ptr_ref):
    k = pl.program_id(0)
    ptr_ref[0] = 0
    ptr_ref[1] = 0
    base = k * blk

    def body(j, carry):
        c = cd_sm[base + j]

        @pl.when((c == 4 * d) | (c == 4 * d + 1))
        def _():
            p = ptr_ref[0]
            ml_ref[pl.ds(base + p, 1), :] = jnp.full((1, 1), j, jnp.int32)
            ptr_ref[0] = p + 1

        @pl.when(c == 4 * d + 3)
        def _():
            p = ptr_ref[1]
            hl_ref[pl.ds(base + p, 1), :] = jnp.full((1, 1), j, jnp.int32)
            ptr_ref[1] = p + 1

        return carry

    jax.lax.fori_loop(0, blk, body, 0)
    mcnt_ref[pl.ds(k, 1), :] = jnp.full((1, 1), ptr_ref[0], jnp.int32)
    hcnt_ref[pl.ds(k, 1), :] = jnp.full((1, 1), ptr_ref[1], jnp.int32)


def _bm_body(d, n, nblk, blk, parents_sm, cd_sm, ml_sm, mcnt_sm, y_ref,
             m_ref, lc_ref):
    k = pl.program_id(0)

    @pl.when(k == 0)
    def _():
        m_ref[...] = jnp.zeros_like(m_ref)
        lc_ref[...] = jnp.zeros_like(lc_ref)

    base = k * blk
    lbase = base
    nb = mcnt_sm[k]

    def body(jj, carry):
        j = ml_sm[lbase + jj]
        i = base + j
        idx = parents_sm[i]
        m_ref[pl.ds(idx, 1), :] += y_ref[pl.ds(j, 1), :]

        @pl.when(cd_sm[i] == 4 * d)
        def _():
            lc_ref[pl.ds(idx, 1), :] += 1.0

        return carry

    jax.lax.fori_loop(0, nb, body, 0)


def _bz_body(d, n, nblk, blk, parents_sm, cd_sm, hl_sm, hcnt_sm, z_ref,
             zacc_ref, hc_ref):
    k = pl.program_id(0)

    @pl.when(k == 0)
    def _():
        zacc_ref[...] = jnp.zeros_like(zacc_ref)
        hc_ref[...] = jnp.zeros_like(hc_ref)

    base = k * blk
    lbase = base
    nb = hcnt_sm[k]

    def body(jj, carry):
        j = hl_sm[lbase + jj]
        idx = parents_sm[base + j]
        zacc_ref[pl.ds(idx, 1), :] += z_ref[pl.ds(j, 1), :]
        hc_ref[pl.ds(idx, 1), :] += 1.0
        return carry

    jax.lax.fori_loop(0, nb, body, 0)


def _c_body(x_ref, pef_ref, m_ref, zacc_ref, lc_ref, hc_ref, w0c_ref, b0_ref,
            w1_ref, b1_ref, v0a_ref, c0_ref, v1_ref, c1_ref, out_ref):
    xb = x_ref[...]
    pre1 = (m_ref[...]
            + jnp.dot(pef_ref[...], w0c_ref[...],
                      preferred_element_type=jnp.float32) + b0_ref[...])
    x_par = (jnp.dot(jnp.maximum(pre1, 0.0), w1_ref[...],
                     preferred_element_type=jnp.float32) + b1_ref[...])
    desg = hc_ref[...] != 0.0
    pm = lc_ref[...] != 0.0
    xd = jnp.where(desg, xb, 0.0)
    pre2 = (jnp.dot(xd, v0a_ref[...], preferred_element_type=jnp.float32)
            + zacc_ref[...] + c0_ref[...])
    x_mer = (jnp.dot(jnp.maximum(pre2, 0.0), v1_ref[...],
                     preferred_element_type=jnp.float32) + c1_ref[...])
    out_ref[...] = jnp.where(pm, x_par, jnp.where(desg, x_mer, xb))


def _row_spec(h):
    return pl.BlockSpec((_BLK, h), lambda k: (k, 0))


def _col_spec():
    return pl.BlockSpec((_BLK, 1), lambda k: (k, 0))


def _full_spec(shape):
    return pl.BlockSpec(shape, lambda k: tuple(0 for _ in shape))


def kernel(x, edge_index, depths, states, parent_edge_features,
           parent_light_edge_features, merger_params, lep_params, lem_params):
    n, h = x.shape
    nblk = n // _BLK
    (w0, b0), (w1, b1) = merger_params
    (u0w, u0b), (u1w, u1b) = lep_params
    (v0w, c0), (v1w, c1) = lem_params
    w0a, w0b, w0c = w0[:h], w0[h:2 * h], w0[2 * h:]
    u0a, u0bb = u0w[:h], u0w[h:]
    v0a, v0b = v0w[:h], v0w[h:]

    parents = jnp.zeros((n,), dtype=edge_index.dtype).at[edge_index[0]].set(
        edge_index[1])
    cd = (depths * 4 + states).astype(jnp.int32)
    dep2 = depths.reshape(n, 1)
    st2 = states.reshape(n, 1)

    f32 = jnp.float32
    i32 = jnp.int32
    wspec = _full_spec((h, h))
    bspec = _full_spec((h,))

    lists = {}
    for d in (3, 2, 1):
        ml_d, hl_d, mcnt_d, hcnt_d = pl.pallas_call(
            functools.partial(_compact_body, d, _BLK),
            grid_spec=pltpu.PrefetchScalarGridSpec(
                num_scalar_prefetch=1,
                grid=(nblk,),
                in_specs=[],
                out_specs=[pl.BlockSpec((n, 1), lambda k, *_: (0, 0)),
                           pl.BlockSpec((n, 1), lambda k, *_: (0, 0)),
                           pl.BlockSpec((nblk, 1), lambda k, *_: (0, 0)),
                           pl.BlockSpec((nblk, 1), lambda k, *_: (0, 0))],
                scratch_shapes=[pltpu.SMEM((8,), i32)],
            ),
            out_shape=[jax.ShapeDtypeStruct((n, 1), i32),
                       jax.ShapeDtypeStruct((n, 1), i32),
                       jax.ShapeDtypeStruct((nblk, 1), i32),
                       jax.ShapeDtypeStruct((nblk, 1), i32)],
        )(cd)
        lists[d] = (ml_d.reshape(n), hl_d.reshape(n),
                    mcnt_d.reshape(nblk), hcnt_d.reshape(nblk))

    for d in (3, 2, 1):
        ml, hl, mcnt, hcnt = lists[d]
        a_call = pl.pallas_call(
            functools.partial(_a_body, d),
            grid=(nblk,),
            in_specs=[_row_spec(h), _row_spec(h), _col_spec(), _col_spec(),
                      wspec, wspec, wspec, wspec, bspec, wspec, bspec, wspec],
            out_specs=[_row_spec(h), _row_spec(h)],
            out_shape=[jax.ShapeDtypeStruct((n, h), f32),
                       jax.ShapeDtypeStruct((n, h), f32)],
        )
        y, z = a_call(x, parent_light_edge_features, dep2, st2,
                      w0a, w0b, u0a, u0bb, u0b, u1w, u1b, v0b)

        def _scatter_spec():
            return pltpu.PrefetchScalarGridSpec(
                num_scalar_prefetch=4,
                grid=(nblk,),
                in_specs=[pl.BlockSpec((_BLK, h), lambda k, *_: (k, 0))],
                out_specs=[pl.BlockSpec((n, h), lambda k, *_: (0, 0)),
                           pl.BlockSpec((n, 1), lambda k, *_: (0, 0))],
            )

        m, lc = pl.pallas_call(
            functools.partial(_bm_body, d, n, nblk, _BLK),
            grid_spec=_scatter_spec(),
            out_shape=[jax.ShapeDtypeStruct((n, h), f32),
                       jax.ShapeDtypeStruct((n, 1), f32)],
        )(parents, cd, ml, mcnt, y)
        zacc, hc = pl.pallas_call(
            functools.partial(_bz_body, d, n, nblk, _BLK),
            grid_spec=_scatter_spec(),
            out_shape=[jax.ShapeDtypeStruct((n, h), f32),
                       jax.ShapeDtypeStruct((n, 1), f32)],
        )(parents, cd, hl, hcnt, z)

        c_call = pl.pallas_call(
            _c_body,
            grid=(nblk,),
            in_specs=[_row_spec(h), _row_spec(h), _row_spec(h), _row_spec(h),
                      _col_spec(), _col_spec(),
                      wspec, bspec, wspec, bspec, wspec, bspec, wspec, bspec],
            out_specs=_row_spec(h),
            out_shape=jax.ShapeDtypeStruct((n, h), f32),
        )
        x = c_call(x, parent_edge_features, m, zacc, lc, hc,
                   w0c, b0, w1, b1, v0a, c0, v1w, c1)
    return x


# speedup vs baseline: 1.5923x; 1.1462x over previous
import functools

import jax
import jax.numpy as jnp
from jax.experimental import pallas as pl
from jax.experimental.pallas import tpu as pltpu

_BLK = 2000


def _a_body(d, x_ref, plef_ref, dep_ref, st_ref, w0a_ref, w0b_ref, u0a_ref,
            u0b_ref, u0_ref, u1w_ref, u1b_ref, v0b_ref, y_ref, z_ref):
    dep = dep_ref[...]
    st = st_ref[...]
    xb = x_ref[...]
    am = ((dep == d) & (st == 0)).astype(jnp.float32)
    bm = ((dep == d) & (st == 1)).astype(jnp.float32)
    hm = ((dep == d) & (st == 3)).astype(jnp.float32)
    y_ref[...] = (
        jnp.dot(xb * am, w0a_ref[...], preferred_element_type=jnp.float32)
        + jnp.dot(xb * bm, w0b_ref[...], preferred_element_type=jnp.float32))
    lh = jnp.maximum(
        jnp.dot(xb, u0a_ref[...], preferred_element_type=jnp.float32)
        + jnp.dot(plef_ref[...], u0b_ref[...], preferred_element_type=jnp.float32)
        + u0_ref[...], 0.0)
    ph = (jnp.dot(lh, u1w_ref[...], preferred_element_type=jnp.float32)
          + u1b_ref[...]) * hm
    z_ref[...] = jnp.dot(ph, v0b_ref[...], preferred_element_type=jnp.float32)


def _compact_body(d, blk, cd_sm, ml_ref, hl_ref, mcnt_ref, hcnt_ref, ptr_ref):
    k = pl.program_id(0)
    ptr_ref[0] = 0
    ptr_ref[1] = 0
    base = k * blk

    def body(jo, carry):
        jb = jo * 4
        for u in range(4):
            j = jb + u
            c = cd_sm[base + j]

            @pl.when((c == 4 * d) | (c == 4 * d + 1))
            def _():
                p = ptr_ref[0]
                ml_ref[pl.ds(base + p, 1), :] = jnp.full((1, 1), j, jnp.int32)
                ptr_ref[0] = p + 1

            @pl.when(c == 4 * d + 3)
            def _():
                p = ptr_ref[1]
                hl_ref[pl.ds(base + p, 1), :] = jnp.full((1, 1), j, jnp.int32)
                ptr_ref[1] = p + 1

        return carry

    jax.lax.fori_loop(0, blk // 4, body, 0)
    mcnt_ref[pl.ds(k, 1), :] = jnp.full((1, 1), ptr_ref[0], jnp.int32)
    hcnt_ref[pl.ds(k, 1), :] = jnp.full((1, 1), ptr_ref[1], jnp.int32)


def _bm_body(d, n, nblk, blk, parents_sm, cd_sm, ml_sm, mcnt_sm, y_ref,
             m_ref, lc_ref):
    k = pl.program_id(0)

    @pl.when(k == 0)
    def _():
        m_ref[...] = jnp.zeros_like(m_ref)
        lc_ref[...] = jnp.zeros_like(lc_ref)

    base = k * blk
    lbase = base
    nb = mcnt_sm[k]

    def body(jj, carry):
        j = ml_sm[lbase + jj]
        i = base + j
        idx = parents_sm[i]
        m_ref[pl.ds(idx, 1), :] += y_ref[pl.ds(j, 1), :]

        @pl.when(cd_sm[i] == 4 * d)
        def _():
            lc_ref[pl.ds(idx, 1), :] += 1.0

        return carry

    jax.lax.fori_loop(0, nb, body, 0)


def _bz_body(d, n, nblk, blk, parents_sm, cd_sm, hl_sm, hcnt_sm, z_ref,
             zacc_ref, hc_ref):
    k = pl.program_id(0)

    @pl.when(k == 0)
    def _():
        zacc_ref[...] = jnp.zeros_like(zacc_ref)
        hc_ref[...] = jnp.zeros_like(hc_ref)

    base = k * blk
    lbase = base
    nb = hcnt_sm[k]

    def body(jj, carry):
        j = hl_sm[lbase + jj]
        idx = parents_sm[base + j]
        zacc_ref[pl.ds(idx, 1), :] += z_ref[pl.ds(j, 1), :]
        hc_ref[pl.ds(idx, 1), :] += 1.0
        return carry

    jax.lax.fori_loop(0, nb, body, 0)


def _c_body(x_ref, pef_ref, m_ref, zacc_ref, lc_ref, hc_ref, w0c_ref, b0_ref,
            w1_ref, b1_ref, v0a_ref, c0_ref, v1_ref, c1_ref, out_ref):
    xb = x_ref[...]
    pre1 = (m_ref[...]
            + jnp.dot(pef_ref[...], w0c_ref[...],
                      preferred_element_type=jnp.float32) + b0_ref[...])
    x_par = (jnp.dot(jnp.maximum(pre1, 0.0), w1_ref[...],
                     preferred_element_type=jnp.float32) + b1_ref[...])
    desg = hc_ref[...] != 0.0
    pm = lc_ref[...] != 0.0
    xd = jnp.where(desg, xb, 0.0)
    pre2 = (jnp.dot(xd, v0a_ref[...], preferred_element_type=jnp.float32)
            + zacc_ref[...] + c0_ref[...])
    x_mer = (jnp.dot(jnp.maximum(pre2, 0.0), v1_ref[...],
                     preferred_element_type=jnp.float32) + c1_ref[...])
    out_ref[...] = jnp.where(pm, x_par, jnp.where(desg, x_mer, xb))


def _row_spec(h):
    return pl.BlockSpec((_BLK, h), lambda k: (k, 0))


def _col_spec():
    return pl.BlockSpec((_BLK, 1), lambda k: (k, 0))


def _full_spec(shape):
    return pl.BlockSpec(shape, lambda k: tuple(0 for _ in shape))


def kernel(x, edge_index, depths, states, parent_edge_features,
           parent_light_edge_features, merger_params, lep_params, lem_params):
    n, h = x.shape
    nblk = n // _BLK
    (w0, b0), (w1, b1) = merger_params
    (u0w, u0b), (u1w, u1b) = lep_params
    (v0w, c0), (v1w, c1) = lem_params
    w0a, w0b, w0c = w0[:h], w0[h:2 * h], w0[2 * h:]
    u0a, u0bb = u0w[:h], u0w[h:]
    v0a, v0b = v0w[:h], v0w[h:]

    parents = jnp.zeros((n,), dtype=edge_index.dtype).at[edge_index[0]].set(
        edge_index[1])
    cd = (depths * 4 + states).astype(jnp.int32)
    dep2 = depths.reshape(n, 1)
    st2 = states.reshape(n, 1)

    f32 = jnp.float32
    i32 = jnp.int32
    wspec = _full_spec((h, h))
    bspec = _full_spec((h,))

    lists = {}
    for d in (3, 2, 1):
        ml_d, hl_d, mcnt_d, hcnt_d = pl.pallas_call(
            functools.partial(_compact_body, d, _BLK),
            grid_spec=pltpu.PrefetchScalarGridSpec(
                num_scalar_prefetch=1,
                grid=(nblk,),
                in_specs=[],
                out_specs=[pl.BlockSpec((n, 1), lambda k, *_: (0, 0)),
                           pl.BlockSpec((n, 1), lambda k, *_: (0, 0)),
                           pl.BlockSpec((nblk, 1), lambda k, *_: (0, 0)),
                           pl.BlockSpec((nblk, 1), lambda k, *_: (0, 0))],
                scratch_shapes=[pltpu.SMEM((8,), i32)],
            ),
            out_shape=[jax.ShapeDtypeStruct((n, 1), i32),
                       jax.ShapeDtypeStruct((n, 1), i32),
                       jax.ShapeDtypeStruct((nblk, 1), i32),
                       jax.ShapeDtypeStruct((nblk, 1), i32)],
        )(cd)
        lists[d] = (ml_d.reshape(n), hl_d.reshape(n),
                    mcnt_d.reshape(nblk), hcnt_d.reshape(nblk))

    for d in (3, 2, 1):
        ml, hl, mcnt, hcnt = lists[d]
        a_call = pl.pallas_call(
            functools.partial(_a_body, d),
            grid=(nblk,),
            in_specs=[_row_spec(h), _row_spec(h), _col_spec(), _col_spec(),
                      wspec, wspec, wspec, wspec, bspec, wspec, bspec, wspec],
            out_specs=[_row_spec(h), _row_spec(h)],
            out_shape=[jax.ShapeDtypeStruct((n, h), f32),
                       jax.ShapeDtypeStruct((n, h), f32)],
        )
        y, z = a_call(x, parent_light_edge_features, dep2, st2,
                      w0a, w0b, u0a, u0bb, u0b, u1w, u1b, v0b)

        def _scatter_spec():
            return pltpu.PrefetchScalarGridSpec(
                num_scalar_prefetch=4,
                grid=(nblk,),
                in_specs=[pl.BlockSpec((_BLK, h), lambda k, *_: (k, 0))],
                out_specs=[pl.BlockSpec((n, h), lambda k, *_: (0, 0)),
                           pl.BlockSpec((n, 1), lambda k, *_: (0, 0))],
            )

        m, lc = pl.pallas_call(
            functools.partial(_bm_body, d, n, nblk, _BLK),
            grid_spec=_scatter_spec(),
            out_shape=[jax.ShapeDtypeStruct((n, h), f32),
                       jax.ShapeDtypeStruct((n, 1), f32)],
        )(parents, cd, ml, mcnt, y)
        zacc, hc = pl.pallas_call(
            functools.partial(_bz_body, d, n, nblk, _BLK),
            grid_spec=_scatter_spec(),
            out_shape=[jax.ShapeDtypeStruct((n, h), f32),
                       jax.ShapeDtypeStruct((n, 1), f32)],
        )(parents, cd, hl, hcnt, z)

        c_call = pl.pallas_call(
            _c_body,
            grid=(nblk,),
            in_specs=[_row_spec(h), _row_spec(h), _row_spec(h), _row_spec(h),
                      _col_spec(), _col_spec(),
                      wspec, bspec, wspec, bspec, wspec, bspec, wspec, bspec],
            out_specs=_row_spec(h),
            out_shape=jax.ShapeDtypeStruct((n, h), f32),
        )
        x = c_call(x, parent_edge_features, m, zacc, lc, hc,
                   w0c, b0, w1, b1, v0a, c0, v1w, c1)
    return x


# speedup vs baseline: 1.6412x; 1.0307x over previous
import functools

import jax
import jax.numpy as jnp
from jax.experimental import pallas as pl
from jax.experimental.pallas import tpu as pltpu

_BLK = 2000


def _a_body(d, x_ref, plef_ref, dep_ref, st_ref, w0a_ref, w0b_ref, u0a_ref,
            u0b_ref, u0_ref, u1w_ref, u1b_ref, v0b_ref, y_ref, z_ref):
    dep = dep_ref[...]
    st = st_ref[...]
    xb = x_ref[...]
    am = ((dep == d) & (st == 0)).astype(jnp.float32)
    bm = ((dep == d) & (st == 1)).astype(jnp.float32)
    hm = ((dep == d) & (st == 3)).astype(jnp.float32)
    y_ref[...] = (
        jnp.dot(xb * am, w0a_ref[...], preferred_element_type=jnp.float32)
        + jnp.dot(xb * bm, w0b_ref[...], preferred_element_type=jnp.float32))
    lh = jnp.maximum(
        jnp.dot(xb, u0a_ref[...], preferred_element_type=jnp.float32)
        + jnp.dot(plef_ref[...], u0b_ref[...], preferred_element_type=jnp.float32)
        + u0_ref[...], 0.0)
    ph = (jnp.dot(lh, u1w_ref[...], preferred_element_type=jnp.float32)
          + u1b_ref[...]) * hm
    z_ref[...] = jnp.dot(ph, v0b_ref[...], preferred_element_type=jnp.float32)


def _compact_body(d, blk, cd_sm, ml_ref, hl_ref, mcnt_ref, hcnt_ref, ptr_ref):
    k = pl.program_id(0)
    ptr_ref[0] = 0
    ptr_ref[1] = 0
    base = k * blk

    def body(jo, carry):
        jb = jo * 8
        for u in range(8):
            j = jb + u
            c = cd_sm[base + j]

            @pl.when((c == 4 * d) | (c == 4 * d + 1))
            def _():
                p = ptr_ref[0]
                ml_ref[pl.ds(base + p, 1), :] = jnp.full((1, 1), j, jnp.int32)
                ptr_ref[0] = p + 1

            @pl.when(c == 4 * d + 3)
            def _():
                p = ptr_ref[1]
                hl_ref[pl.ds(base + p, 1), :] = jnp.full((1, 1), j, jnp.int32)
                ptr_ref[1] = p + 1

        return carry

    jax.lax.fori_loop(0, blk // 8, body, 0)
    mcnt_ref[pl.ds(k, 1), :] = jnp.full((1, 1), ptr_ref[0], jnp.int32)
    hcnt_ref[pl.ds(k, 1), :] = jnp.full((1, 1), ptr_ref[1], jnp.int32)


def _bm_body(d, n, nblk, blk, parents_sm, cd_sm, ml_sm, mcnt_sm, y_ref,
             m_ref, lc_ref):
    k = pl.program_id(0)

    @pl.when(k == 0)
    def _():
        m_ref[...] = jnp.zeros_like(m_ref)
        lc_ref[...] = jnp.zeros_like(lc_ref)

    base = k * blk
    lbase = base
    nb = mcnt_sm[k]

    def body(jj, carry):
        j = ml_sm[lbase + jj]
        i = base + j
        idx = parents_sm[i]
        m_ref[pl.ds(idx, 1), :] += y_ref[pl.ds(j, 1), :]

        @pl.when(cd_sm[i] == 4 * d)
        def _():
            lc_ref[pl.ds(idx, 1), :] += 1.0

        return carry

    jax.lax.fori_loop(0, nb, body, 0)


def _bz_body(d, n, nblk, blk, parents_sm, cd_sm, hl_sm, hcnt_sm, z_ref,
             zacc_ref, hc_ref):
    k = pl.program_id(0)

    @pl.when(k == 0)
    def _():
        zacc_ref[...] = jnp.zeros_like(zacc_ref)
        hc_ref[...] = jnp.zeros_like(hc_ref)

    base = k * blk
    lbase = base
    nb = hcnt_sm[k]

    def body(jj, carry):
        j = hl_sm[lbase + jj]
        idx = parents_sm[base + j]
        zacc_ref[pl.ds(idx, 1), :] += z_ref[pl.ds(j, 1), :]
        hc_ref[pl.ds(idx, 1), :] += 1.0
        return carry

    jax.lax.fori_loop(0, nb, body, 0)


def _c_body(x_ref, pef_ref, m_ref, zacc_ref, lc_ref, hc_ref, w0c_ref, b0_ref,
            w1_ref, b1_ref, v0a_ref, c0_ref, v1_ref, c1_ref, out_ref):
    xb = x_ref[...]
    pre1 = (m_ref[...]
            + jnp.dot(pef_ref[...], w0c_ref[...],
                      preferred_element_type=jnp.float32) + b0_ref[...])
    x_par = (jnp.dot(jnp.maximum(pre1, 0.0), w1_ref[...],
                     preferred_element_type=jnp.float32) + b1_ref[...])
    desg = hc_ref[...] != 0.0
    pm = lc_ref[...] != 0.0
    xd = jnp.where(desg, xb, 0.0)
    pre2 = (jnp.dot(xd, v0a_ref[...], preferred_element_type=jnp.float32)
            + zacc_ref[...] + c0_ref[...])
    x_mer = (jnp.dot(jnp.maximum(pre2, 0.0), v1_ref[...],
                     preferred_element_type=jnp.float32) + c1_ref[...])
    out_ref[...] = jnp.where(pm, x_par, jnp.where(desg, x_mer, xb))


def _row_spec(h):
    return pl.BlockSpec((_BLK, h), lambda k: (k, 0))


def _col_spec():
    return pl.BlockSpec((_BLK, 1), lambda k: (k, 0))


def _full_spec(shape):
    return pl.BlockSpec(shape, lambda k: tuple(0 for _ in shape))


def kernel(x, edge_index, depths, states, parent_edge_features,
           parent_light_edge_features, merger_params, lep_params, lem_params):
    n, h = x.shape
    nblk = n // _BLK
    (w0, b0), (w1, b1) = merger_params
    (u0w, u0b), (u1w, u1b) = lep_params
    (v0w, c0), (v1w, c1) = lem_params
    w0a, w0b, w0c = w0[:h], w0[h:2 * h], w0[2 * h:]
    u0a, u0bb = u0w[:h], u0w[h:]
    v0a, v0b = v0w[:h], v0w[h:]

    parents = jnp.zeros((n,), dtype=edge_index.dtype).at[edge_index[0]].set(
        edge_index[1])
    cd = (depths * 4 + states).astype(jnp.int32)
    dep2 = depths.reshape(n, 1)
    st2 = states.reshape(n, 1)

    f32 = jnp.float32
    i32 = jnp.int32
    wspec = _full_spec((h, h))
    bspec = _full_spec((h,))

    lists = {}
    for d in (3, 2, 1):
        ml_d, hl_d, mcnt_d, hcnt_d = pl.pallas_call(
            functools.partial(_compact_body, d, _BLK),
            grid_spec=pltpu.PrefetchScalarGridSpec(
                num_scalar_prefetch=1,
                grid=(nblk,),
                in_specs=[],
                out_specs=[pl.BlockSpec((n, 1), lambda k, *_: (0, 0)),
                           pl.BlockSpec((n, 1), lambda k, *_: (0, 0)),
                           pl.BlockSpec((nblk, 1), lambda k, *_: (0, 0)),
                           pl.BlockSpec((nblk, 1), lambda k, *_: (0, 0))],
                scratch_shapes=[pltpu.SMEM((8,), i32)],
            ),
            out_shape=[jax.ShapeDtypeStruct((n, 1), i32),
                       jax.ShapeDtypeStruct((n, 1), i32),
                       jax.ShapeDtypeStruct((nblk, 1), i32),
                       jax.ShapeDtypeStruct((nblk, 1), i32)],
        )(cd)
        lists[d] = (ml_d.reshape(n), hl_d.reshape(n),
                    mcnt_d.reshape(nblk), hcnt_d.reshape(nblk))

    for d in (3, 2, 1):
        ml, hl, mcnt, hcnt = lists[d]
        a_call = pl.pallas_call(
            functools.partial(_a_body, d),
            grid=(nblk,),
            in_specs=[_row_spec(h), _row_spec(h), _col_spec(), _col_spec(),
                      wspec, wspec, wspec, wspec, bspec, wspec, bspec, wspec],
            out_specs=[_row_spec(h), _row_spec(h)],
            out_shape=[jax.ShapeDtypeStruct((n, h), f32),
                       jax.ShapeDtypeStruct((n, h), f32)],
        )
        y, z = a_call(x, parent_light_edge_features, dep2, st2,
                      w0a, w0b, u0a, u0bb, u0b, u1w, u1b, v0b)

        def _scatter_spec():
            return pltpu.PrefetchScalarGridSpec(
                num_scalar_prefetch=4,
                grid=(nblk,),
                in_specs=[pl.BlockSpec((_BLK, h), lambda k, *_: (k, 0))],
                out_specs=[pl.BlockSpec((n, h), lambda k, *_: (0, 0)),
                           pl.BlockSpec((n, 1), lambda k, *_: (0, 0))],
            )

        m, lc = pl.pallas_call(
            functools.partial(_bm_body, d, n, nblk, _BLK),
            grid_spec=_scatter_spec(),
            out_shape=[jax.ShapeDtypeStruct((n, h), f32),
                       jax.ShapeDtypeStruct((n, 1), f32)],
        )(parents, cd, ml, mcnt, y)
        zacc, hc = pl.pallas_call(
            functools.partial(_bz_body, d, n, nblk, _BLK),
            grid_spec=_scatter_spec(),
            out_shape=[jax.ShapeDtypeStruct((n, h), f32),
                       jax.ShapeDtypeStruct((n, 1), f32)],
        )(parents, cd, hl, hcnt, z)

        c_call = pl.pallas_call(
            _c_body,
            grid=(nblk,),
            in_specs=[_row_spec(h), _row_spec(h), _row_spec(h), _row_spec(h),
                      _col_spec(), _col_spec(),
                      wspec, bspec, wspec, bspec, wspec, bspec, wspec, bspec],
            out_specs=_row_spec(h),
            out_shape=jax.ShapeDtypeStruct((n, h), f32),
        )
        x = c_call(x, parent_edge_features, m, zacc, lc, hc,
                   w0c, b0, w1, b1, v0a, c0, v1w, c1)
    return x
